# baseline (device time: 17477 ns/iter reference)
import jax
import jax.numpy as jnp
from jax import lax
from jax.experimental import pallas as pl
from jax.experimental.pallas import tpu as pltpu

N_DEV = 4
M = 512
H = M // 2
N_PER = 512

FWDA, FWDB, DIRA, DIRB, SUMA, SUMB = range(6)


def kernel(x):
    def body(x_hbm, out_ref, stgO, stgR, stgL, sOppA, sOppB, sDirA, sDirB,
             sA, sB, rbuf, send_sems, recv_sems, local_sems):
        my = lax.axis_index("i")
        left = (my - 1) % N_DEV
        right = (my + 1) % N_DEV
        opp = (my + 2) % N_DEV

        def fetch(c, dst, sem_ix):
            return pltpu.make_async_copy(
                x_hbm.at[0, :, pl.ds(c * N_PER, N_PER)], dst,
                local_sems.at[sem_ix],
            )

        cpO = fetch(opp, stgO, 0)
        cpO.start()

        barrier_sem = pltpu.get_barrier_semaphore()
        for nbr in [left, right]:
            pl.semaphore_signal(
                barrier_sem, inc=1,
                device_id=(nbr,), device_id_type=pl.DeviceIdType.MESH,
            )
        pl.semaphore_wait(barrier_sem, 2)

        def remote(src, dst_ix, target):
            return pltpu.make_async_remote_copy(
                src_ref=src, dst_ref=rbuf.at[dst_ix],
                send_sem=send_sems.at[dst_ix], recv_sem=recv_sems.at[dst_ix],
                device_id=(target,), device_id_type=pl.DeviceIdType.MESH,
            )

        fwdA = remote(sOppA.at[:, :], FWDA, left)
        fwdB = remote(sOppB.at[:, :], FWDB, right)
        dirA = remote(sDirA.at[:, :], DIRA, right)
        dirB = remote(sDirB.at[:, :], DIRB, left)
        sumA = remote(sA.at[:, :], SUMA, left)
        sumB = remote(sB.at[:, :], SUMB, right)

        cpO.wait()
        cpR = fetch(right, stgR, 1)
        cpR.start()
        cpL = fetch(left, stgL, 2)
        cpL.start()
        cpMy = pltpu.make_async_copy(
            x_hbm.at[0, :, pl.ds(my * N_PER, N_PER)], out_ref,
            local_sems.at[3],
        )
        cpMy.start()

        sOppA[...] = stgO[0:H, :].astype(jnp.bfloat16)
        fwdA.start()
        sOppB[...] = stgO[H:M, :].astype(jnp.bfloat16)
        fwdB.start()

        cpR.wait()
        sDirA[...] = stgR[0:H, :].astype(jnp.bfloat16)
        dirA.start()
        cpL.wait()
        sDirB[...] = stgL[H:M, :].astype(jnp.bfloat16)
        dirB.start()

        fwdA.wait_recv()
        sA[...] = rbuf[FWDA] + stgL[0:H, :].astype(jnp.bfloat16)
        sumA.start()
        fwdB.wait_recv()
        sB[...] = rbuf[FWDB] + stgR[H:M, :].astype(jnp.bfloat16)
        sumB.start()

        cpMy.wait()
        dirA.wait_recv()
        sumA.wait_recv()
        out_ref[0:H, :] = (
            out_ref[0:H, :]
            + rbuf[DIRA].astype(jnp.float32)
            + rbuf[SUMA].astype(jnp.float32)
        )
        dirB.wait_recv()
        sumB.wait_recv()
        out_ref[H:M, :] = (
            out_ref[H:M, :]
            + rbuf[DIRB].astype(jnp.float32)
            + rbuf[SUMB].astype(jnp.float32)
        )

        for r in (fwdA, fwdB, dirA, dirB, sumA, sumB):
            r.wait_send()

    return pl.pallas_call(
        body,
        out_shape=jax.ShapeDtypeStruct((M, N_PER), jnp.float32),
        in_specs=[pl.BlockSpec(memory_space=pl.ANY)],
        out_specs=pl.BlockSpec(memory_space=pltpu.VMEM),
        scratch_shapes=[
            pltpu.VMEM((M, N_PER), jnp.float32),
            pltpu.VMEM((M, N_PER), jnp.float32),
            pltpu.VMEM((M, N_PER), jnp.float32),
            pltpu.VMEM((H, N_PER), jnp.bfloat16),
            pltpu.VMEM((H, N_PER), jnp.bfloat16),
            pltpu.VMEM((H, N_PER), jnp.bfloat16),
            pltpu.VMEM((H, N_PER), jnp.bfloat16),
            pltpu.VMEM((H, N_PER), jnp.bfloat16),
            pltpu.VMEM((H, N_PER), jnp.bfloat16),
            pltpu.VMEM((6, H, N_PER), jnp.bfloat16),
            pltpu.SemaphoreType.DMA((6,)),
            pltpu.SemaphoreType.DMA((6,)),
            pltpu.SemaphoreType.DMA((4,)),
        ],
        compiler_params=pltpu.CompilerParams(collective_id=0),
    )(x)


# device time: 16495 ns/iter; 1.0595x vs baseline; 1.0595x over previous
import jax
import jax.numpy as jnp
from jax import lax
from jax.experimental import pallas as pl
from jax.experimental.pallas import tpu as pltpu

N_DEV = 4
M = 512
H = M // 2
N_PER = 512

FWDA, FWDB, DIRA, DIRB, SUMA, SUMB = range(6)


def kernel(x):
    def body(x_ref, out_ref, sOppA, sOppB, sDirA, sDirB, sA, sB, rbuf,
             send_sems, recv_sems):
        my = lax.axis_index("i")
        left = (my - 1) % N_DEV
        right = (my + 1) % N_DEV
        opp = (my + 2) % N_DEV

        barrier_sem = pltpu.get_barrier_semaphore()
        for nbr in [left, right]:
            pl.semaphore_signal(
                barrier_sem, inc=1,
                device_id=(nbr,), device_id_type=pl.DeviceIdType.MESH,
            )
        pl.semaphore_wait(barrier_sem, 2)

        def remote(src, dst_ix, target):
            return pltpu.make_async_remote_copy(
                src_ref=src, dst_ref=rbuf.at[dst_ix],
                send_sem=send_sems.at[dst_ix], recv_sem=recv_sems.at[dst_ix],
                device_id=(target,), device_id_type=pl.DeviceIdType.MESH,
            )

        fwdA = remote(sOppA.at[:, :], FWDA, left)
        fwdB = remote(sOppB.at[:, :], FWDB, right)
        dirA = remote(sDirA.at[:, :], DIRA, right)
        dirB = remote(sDirB.at[:, :], DIRB, left)
        sumA = remote(sA.at[:, :], SUMA, left)
        sumB = remote(sB.at[:, :], SUMB, right)

        def chunk(c, r0, rn):
            return x_ref[0, r0:r0 + rn, c * N_PER:(c + 1) * N_PER]

        for c in range(N_DEV):
            @pl.when(opp == c)
            def _(c=c):
                sOppA[...] = chunk(c, 0, H).astype(jnp.bfloat16)
                sOppB[...] = chunk(c, H, H).astype(jnp.bfloat16)
        fwdA.start()
        fwdB.start()

        for c in range(N_DEV):
            @pl.when(right == c)
            def _(c=c):
                sDirA[...] = chunk(c, 0, H).astype(jnp.bfloat16)
        dirA.start()
        for c in range(N_DEV):
            @pl.when(left == c)
            def _(c=c):
                sDirB[...] = chunk(c, H, H).astype(jnp.bfloat16)
        dirB.start()

        fwdA.wait_recv()
        for c in range(N_DEV):
            @pl.when(left == c)
            def _(c=c):
                sA[...] = rbuf[FWDA] + chunk(c, 0, H).astype(jnp.bfloat16)
        sumA.start()

        fwdB.wait_recv()
        for c in range(N_DEV):
            @pl.when(right == c)
            def _(c=c):
                sB[...] = rbuf[FWDB] + chunk(c, H, H).astype(jnp.bfloat16)
        sumB.start()

        dirA.wait_recv()
        sumA.wait_recv()
        for c in range(N_DEV):
            @pl.when(my == c)
            def _(c=c):
                out_ref[0:H, :] = (
                    chunk(c, 0, H).astype(jnp.bfloat16)
                    + rbuf[DIRA]
                    + rbuf[SUMA]
                )
        dirB.wait_recv()
        sumB.wait_recv()
        for c in range(N_DEV):
            @pl.when(my == c)
            def _(c=c):
                out_ref[H:M, :] = (
                    chunk(c, H, H).astype(jnp.bfloat16)
                    + rbuf[DIRB]
                    + rbuf[SUMB]
                )

        for r in (fwdA, fwdB, dirA, dirB, sumA, sumB):
            r.wait_send()

    return pl.pallas_call(
        body,
        out_shape=jax.ShapeDtypeStruct((M, N_PER), jnp.bfloat16),
        in_specs=[pl.BlockSpec(memory_space=pltpu.VMEM)],
        out_specs=pl.BlockSpec(memory_space=pltpu.VMEM),
        scratch_shapes=[
            pltpu.VMEM((H, N_PER), jnp.bfloat16),
            pltpu.VMEM((H, N_PER), jnp.bfloat16),
            pltpu.VMEM((H, N_PER), jnp.bfloat16),
            pltpu.VMEM((H, N_PER), jnp.bfloat16),
            pltpu.VMEM((H, N_PER), jnp.bfloat16),
            pltpu.VMEM((H, N_PER), jnp.bfloat16),
            pltpu.VMEM((6, H, N_PER), jnp.bfloat16),
            pltpu.SemaphoreType.DMA((6,)),
            pltpu.SemaphoreType.DMA((6,)),
        ],
        compiler_params=pltpu.CompilerParams(collective_id=0),
    )(x)


# device time: 16423 ns/iter; 1.0642x vs baseline; 1.0044x over previous
import jax
import jax.numpy as jnp
from jax import lax
from jax.experimental import pallas as pl
from jax.experimental.pallas import tpu as pltpu

N_DEV = 4
M = 512
H = M // 2
N_PER = 512

FWDA, FWDB, DIRA, DIRB, SUMA, SUMB = range(6)


def kernel(x):
    def body(x_ref, out_ref, sOppA, sOppB, sDirA, sDirB, sA, sB, rbuf,
             send_sems, recv_sems):
        my = lax.axis_index("i")
        left = (my - 1) % N_DEV
        right = (my + 1) % N_DEV
        opp = (my + 2) % N_DEV

        barrier_sem = pltpu.get_barrier_semaphore()
        for nbr in [left, right]:
            pl.semaphore_signal(
                barrier_sem, inc=1,
                device_id=(nbr,), device_id_type=pl.DeviceIdType.MESH,
            )

        def remote(src, dst_ix, target):
            return pltpu.make_async_remote_copy(
                src_ref=src, dst_ref=rbuf.at[dst_ix],
                send_sem=send_sems.at[dst_ix], recv_sem=recv_sems.at[dst_ix],
                device_id=(target,), device_id_type=pl.DeviceIdType.MESH,
            )

        fwdA = remote(sOppA.at[:, :], FWDA, left)
        fwdB = remote(sOppB.at[:, :], FWDB, right)
        dirA = remote(sDirA.at[:, :], DIRA, right)
        dirB = remote(sDirB.at[:, :], DIRB, left)
        sumA = remote(sA.at[:, :], SUMA, left)
        sumB = remote(sB.at[:, :], SUMB, right)

        def chunk(c, r0, rn):
            return x_ref[0, r0:r0 + rn, c * N_PER:(c + 1) * N_PER]

        for c in range(N_DEV):
            @pl.when(opp == c)
            def _(c=c):
                sOppA[...] = chunk(c, 0, H).astype(jnp.bfloat16)
                sOppB[...] = chunk(c, H, H).astype(jnp.bfloat16)
        for c in range(N_DEV):
            @pl.when(right == c)
            def _(c=c):
                sDirA[...] = chunk(c, 0, H).astype(jnp.bfloat16)
                sB[...] = chunk(c, H, H).astype(jnp.bfloat16)
        for c in range(N_DEV):
            @pl.when(left == c)
            def _(c=c):
                sDirB[...] = chunk(c, H, H).astype(jnp.bfloat16)
                sA[...] = chunk(c, 0, H).astype(jnp.bfloat16)
        for c in range(N_DEV):
            @pl.when(my == c)
            def _(c=c):
                out_ref[...] = chunk(c, 0, M).astype(jnp.bfloat16)

        pl.semaphore_wait(barrier_sem, 2)

        fwdA.start()
        fwdB.start()
        dirA.start()
        dirB.start()

        fwdA.wait_recv()
        sA[...] = sA[...] + rbuf[FWDA]
        sumA.start()
        fwdB.wait_recv()
        sB[...] = sB[...] + rbuf[FWDB]
        sumB.start()

        dirA.wait_recv()
        sumA.wait_recv()
        out_ref[0:H, :] = out_ref[0:H, :] + rbuf[DIRA] + rbuf[SUMA]
        dirB.wait_recv()
        sumB.wait_recv()
        out_ref[H:M, :] = out_ref[H:M, :] + rbuf[DIRB] + rbuf[SUMB]

        for r in (fwdA, fwdB, dirA, dirB, sumA, sumB):
            r.wait_send()

    return pl.pallas_call(
        body,
        out_shape=jax.ShapeDtypeStruct((M, N_PER), jnp.bfloat16),
        in_specs=[pl.BlockSpec(memory_space=pltpu.VMEM)],
        out_specs=pl.BlockSpec(memory_space=pltpu.VMEM),
        scratch_shapes=[
            pltpu.VMEM((H, N_PER), jnp.bfloat16),
            pltpu.VMEM((H, N_PER), jnp.bfloat16),
            pltpu.VMEM((H, N_PER), jnp.bfloat16),
            pltpu.VMEM((H, N_PER), jnp.bfloat16),
            pltpu.VMEM((H, N_PER), jnp.bfloat16),
            pltpu.VMEM((H, N_PER), jnp.bfloat16),
            pltpu.VMEM((6, H, N_PER), jnp.bfloat16),
            pltpu.SemaphoreType.DMA((6,)),
            pltpu.SemaphoreType.DMA((6,)),
        ],
        compiler_params=pltpu.CompilerParams(collective_id=0),
    )(x)
